# baseline (speedup 1.0000x reference)
import functools

import jax
import jax.numpy as jnp
import numpy as np
from jax import lax
from jax.experimental import pallas as pl
from jax.experimental.pallas import tpu as pltpu
from jax.experimental.pallas import tpu_sc as plsc

N = 10000
IN = 128
OUT = 128
H = 4
C = OUT // H

NC = 2
NS = 16
NW = NC * NS
NPAD = 10016
ROWW = 144
EB = 48
NBLK = 216
EPT = EB * NBLK
EPAD = EPT * NW

ROWS_PER_TILE = NPAD // NS


def _mm_body(x_ref, wl_ref, wr_ref, wres_ref, xl_ref, xr_ref, res_ref):
    x = x_ref[...]
    xl_ref[...] = jnp.dot(x, wl_ref[...], preferred_element_type=jnp.float32)
    xr_ref[...] = jnp.dot(x, wr_ref[...], preferred_element_type=jnp.float32)
    res_ref[...] = jnp.dot(x, wres_ref[...], preferred_element_type=jnp.float32)


def _sc_edge_body(xl_hbm, xr_hbm, src_hbm, dst_hbm, att_hbm, out_hbm,
                  sidx, didxg, didxs, xl_v, xr_v, msg_v, att_v, acc_sh,
                  gsem_l, gsem_r, ssem):
    cid = lax.axis_index("c")
    sid = lax.axis_index("s")
    tid = cid * NS + sid

    zero = jnp.zeros((16,), jnp.float32)

    def zrow(r, carry):
        for b in range(2):
            for cc in range(ROWW // 16):
                msg_v[b][r, pl.ds(cc * 16, 16)] = zero
        return carry

    lax.fori_loop(0, EB, zrow, 0)
    base = sid * ROWS_PER_TILE
    for k in range(ROWS_PER_TILE // EB):
        pltpu.sync_copy(msg_v[0], acc_sh.at[pl.ds(base + k * EB, EB)])
    rem = ROWS_PER_TILE % EB
    if rem:
        pltpu.sync_copy(msg_v[0].at[pl.ds(0, rem)],
                        acc_sh.at[pl.ds(base + (ROWS_PER_TILE // EB) * EB, rem)])
    plsc.subcore_barrier()

    pltpu.sync_copy(att_hbm, att_v)
    avs = [att_v[pl.ds(cc * 16, 16)] for cc in range(8)]
    iota16 = lax.iota(jnp.int32, 16)

    def compute_block(xl_b, xr_b, msg_b):
        @plsc.parallel_loop(0, EB, 1, unroll=4)
        def edge(e):
            xs = [xl_b[e, pl.ds(cc * 16, 16)] for cc in range(8)]
            us = []
            for h in range(H):
                cc = 2 * h
                s0 = xs[cc] + xr_b[e, pl.ds(cc * 16, 16)]
                s1 = xs[cc + 1] + xr_b[e, pl.ds((cc + 1) * 16, 16)]
                t0 = jnp.maximum(s0, 0.2 * s0) * avs[cc]
                t1 = jnp.maximum(s1, 0.2 * s1) * avs[cc + 1]
                us.append(t0 + t1)
            als = [jnp.sum(u) for u in us]
            wvs = [jnp.exp(jnp.full((16,), a, jnp.float32)) for a in als]
            wrow = jnp.zeros((16,), jnp.float32)
            for h in range(H):
                wrow = jnp.where(iota16 == h, wvs[h], wrow)
            for cc in range(8):
                msg_b[e, pl.ds(cc * 16, 16)] = xs[cc] * wvs[cc // 2]
            msg_b[e, pl.ds(128, 16)] = wrow

    for b in range(2):
        pltpu.sync_copy(src_hbm.at[tid, b], sidx[b])
        pltpu.sync_copy(dst_hbm.at[tid, b], didxg[b])
        pltpu.async_copy(xl_hbm.at[sidx[b]], xl_v[b], gsem_l[b])
        pltpu.async_copy(xr_hbm.at[didxg[b]], xr_v[b], gsem_r[b])
        pltpu.sync_copy(dst_hbm.at[tid, NBLK + b], didxs[b])
        pltpu.async_copy(msg_v[b], acc_sh.at[didxs[b]], ssem[b], add=True)

    def blk_pair(k, carry):
        for b in range(2):
            j = 2 * k + b
            cpl = pltpu.make_async_copy(xl_hbm.at[sidx[b]], xl_v[b], gsem_l[b])
            cpr = pltpu.make_async_copy(xr_hbm.at[didxg[b]], xr_v[b], gsem_r[b])
            cpl.wait()
            cpr.wait()
            cps = pltpu.make_async_copy(msg_v[b], acc_sh.at[didxs[b]],
                                        ssem[b])
            cps.wait()
            pltpu.sync_copy(dst_hbm.at[tid, j], didxs[b])
            compute_block(xl_v[b], xr_v[b], msg_v[b])
            pltpu.async_copy(msg_v[b], acc_sh.at[didxs[b]], ssem[b], add=True)
            pltpu.sync_copy(src_hbm.at[tid, j + 2], sidx[b])
            pltpu.sync_copy(dst_hbm.at[tid, j + 2], didxg[b])
            pltpu.async_copy(xl_hbm.at[sidx[b]], xl_v[b], gsem_l[b])
            pltpu.async_copy(xr_hbm.at[didxg[b]], xr_v[b], gsem_r[b])
        return carry

    lax.fori_loop(0, NBLK // 2, blk_pair, 0)

    for b in range(2):
        pltpu.make_async_copy(xl_hbm.at[sidx[b]], xl_v[b], gsem_l[b]).wait()
        pltpu.make_async_copy(xr_hbm.at[didxg[b]], xr_v[b], gsem_r[b]).wait()
        pltpu.make_async_copy(msg_v[b], acc_sh.at[didxs[b]], ssem[b]).wait()
    plsc.subcore_barrier()

    pltpu.sync_copy(acc_sh.at[pl.ds(base, ROWS_PER_TILE)],
                    out_hbm.at[cid, pl.ds(base, ROWS_PER_TILE)])


def _post_body(acc_ref, res_ref, msel_ref, bias_ref, gam_ref, beta_ref, out_ref):
    a = acc_ref[0] + acc_ref[1]
    msg = a[:, :OUT]
    scale = jnp.dot(a, msel_ref[...], preferred_element_type=jnp.float32) + 1e-16
    out = msg / scale + res_ref[...] + bias_ref[...]
    mu = jnp.mean(out, axis=1, keepdims=True)
    var = jnp.mean((out - mu) * (out - mu), axis=1, keepdims=True)
    out = (out - mu) * lax.rsqrt(var + 1e-5) * gam_ref[...] + beta_ref[...]
    out_ref[...] = 0.5 * out * (1.0 + jnp.tanh(
        np.sqrt(2.0 / np.pi) * (out + 0.044715 * out * out * out)))


def kernel(x_in, edge_index, W_l, W_r, att, bias, W_res, ln_gamma, ln_beta):
    x_pad = jnp.concatenate(
        [x_in, jnp.zeros((NPAD - N, IN), jnp.float32)], axis=0)

    mm_rows = 2504
    xl_tab, xr_tab, res = pl.pallas_call(
        _mm_body,
        grid=(NPAD // mm_rows,),
        in_specs=[
            pl.BlockSpec((mm_rows, IN), lambda i: (i, 0)),
            pl.BlockSpec((IN, OUT), lambda i: (0, 0)),
            pl.BlockSpec((IN, OUT), lambda i: (0, 0)),
            pl.BlockSpec((IN, OUT), lambda i: (0, 0)),
        ],
        out_specs=[
            pl.BlockSpec((mm_rows, OUT), lambda i: (i, 0)),
            pl.BlockSpec((mm_rows, OUT), lambda i: (i, 0)),
            pl.BlockSpec((mm_rows, OUT), lambda i: (i, 0)),
        ],
        out_shape=[
            jax.ShapeDtypeStruct((NPAD, OUT), jnp.float32),
            jax.ShapeDtypeStruct((NPAD, OUT), jnp.float32),
            jax.ShapeDtypeStruct((NPAD, OUT), jnp.float32),
        ],
    )(x_pad, W_l, W_r, W_res)

    ei = edge_index.astype(jnp.int32)
    sl = jnp.arange(N, dtype=jnp.int32)
    npad_edges = EPAD - ei.shape[1] - N
    padv = jnp.full((npad_edges,), N, jnp.int32)
    dummy_blocks = jnp.full((NW, 2, EB), N, jnp.int32)
    src = jnp.concatenate([ei[0], sl, padv]).reshape(NW, NBLK, EB)
    dst = jnp.concatenate([ei[1], sl, padv]).reshape(NW, NBLK, EB)
    src = jnp.concatenate([src, dummy_blocks], axis=1)
    dst = jnp.concatenate([dst, dummy_blocks], axis=1)
    att_flat = att.reshape(H * C)

    sc_edge = functools.partial(
        pl.kernel,
        mesh=plsc.VectorSubcoreMesh(core_axis_name="c", subcore_axis_name="s"),
        out_type=jax.ShapeDtypeStruct((NC, NPAD, ROWW), jnp.float32),
        scratch_types=[
            [pltpu.VMEM((EB,), jnp.int32)] * 2,
            [pltpu.VMEM((EB,), jnp.int32)] * 2,
            [pltpu.VMEM((EB,), jnp.int32)] * 2,
            [pltpu.VMEM((EB, IN), jnp.float32)] * 2,
            [pltpu.VMEM((EB, IN), jnp.float32)] * 2,
            [pltpu.VMEM((EB, ROWW), jnp.float32)] * 2,
            pltpu.VMEM((IN,), jnp.float32),
            pltpu.VMEM_SHARED((NPAD, ROWW), jnp.float32),
            [pltpu.SemaphoreType.DMA] * 2,
            [pltpu.SemaphoreType.DMA] * 2,
            [pltpu.SemaphoreType.DMA] * 2,
        ],
        compiler_params=pltpu.CompilerParams(
            use_tc_tiling_on_sc=False, needs_layout_passes=False),
    )(_sc_edge_body)
    acc = sc_edge(xl_tab, xr_tab, src, dst, att_flat)

    msel = np.zeros((ROWW, OUT), np.float32)
    for h in range(H):
        msel[OUT + h, h * C:(h + 1) * C] = 1.0
    msel = jnp.asarray(msel)
    post_rows = 2000
    out = pl.pallas_call(
        _post_body,
        grid=(N // post_rows,),
        in_specs=[
            pl.BlockSpec((NC, post_rows, ROWW), lambda i: (0, i, 0)),
            pl.BlockSpec((post_rows, OUT), lambda i: (i, 0)),
            pl.BlockSpec((ROWW, OUT), lambda i: (0, 0)),
            pl.BlockSpec((1, OUT), lambda i: (0, 0)),
            pl.BlockSpec((1, OUT), lambda i: (0, 0)),
            pl.BlockSpec((1, OUT), lambda i: (0, 0)),
        ],
        out_specs=pl.BlockSpec((post_rows, OUT), lambda i: (i, 0)),
        out_shape=jax.ShapeDtypeStruct((N, OUT), jnp.float32),
    )(acc, res[:N], msel, bias.reshape(1, OUT), ln_gamma.reshape(1, OUT),
      ln_beta.reshape(1, OUT))
    return out


# speedup vs baseline: 1.3645x; 1.3645x over previous
import functools

import jax
import jax.numpy as jnp
import numpy as np
from jax import lax
from jax.experimental import pallas as pl
from jax.experimental.pallas import tpu as pltpu
from jax.experimental.pallas import tpu_sc as plsc

N = 10000
IN = 128
OUT = 128
H = 4
C = OUT // H

NC = 2
NS = 16
NW = NC * NS
NPAD = 10016
ROWW = 144
EB = 36
NBLK = 288
CHUNK = 24
NCHUNK = NBLK // CHUNK
NBLK_PAD = (NCHUNK + 1) * CHUNK
EPT = EB * NBLK
EPAD = EPT * NW

ROWS_PER_TILE = NPAD // NS


def _mm_body(x_ref, wl_ref, wr_ref, wres_ref, xl_ref, xr_ref, res_ref):
    x = x_ref[...]
    xl_ref[...] = jnp.dot(x, wl_ref[...], preferred_element_type=jnp.float32)
    xr_ref[...] = jnp.dot(x, wr_ref[...], preferred_element_type=jnp.float32)
    res_ref[...] = jnp.dot(x, wres_ref[...], preferred_element_type=jnp.float32)


def _sc_edge_body(xl_hbm, xr_hbm, src_hbm, dst_hbm, att_hbm, out_hbm,
                  schunk, dchunk, xl_v, xr_v, msg_v, att_v, acc_sh,
                  gsem_l, gsem_r, ssem):
    cid = lax.axis_index("c")
    sid = lax.axis_index("s")
    tid = cid * NS + sid

    zero = jnp.zeros((16,), jnp.float32)

    def zrow(r, carry):
        for b in range(2):
            for cc in range(ROWW // 16):
                msg_v[b][r, pl.ds(cc * 16, 16)] = zero
        return carry

    lax.fori_loop(0, EB, zrow, 0)
    base = sid * ROWS_PER_TILE
    for k in range(ROWS_PER_TILE // EB):
        pltpu.sync_copy(msg_v[0], acc_sh.at[pl.ds(base + k * EB, EB)])
    rem = ROWS_PER_TILE % EB
    if rem:
        pltpu.sync_copy(msg_v[0].at[pl.ds(0, rem)],
                        acc_sh.at[pl.ds(base + (ROWS_PER_TILE // EB) * EB, rem)])
    plsc.subcore_barrier()

    pltpu.sync_copy(att_hbm, att_v)
    avs = [att_v[pl.ds(cc * 16, 16)] for cc in range(8)]
    iota16 = lax.iota(jnp.int32, 16)

    def compute_block(xl_b, xr_b, msg_b):
        @plsc.parallel_loop(0, EB, 1, unroll=2)
        def edge(e):
            xs = [xl_b[e, pl.ds(cc * 16, 16)] for cc in range(8)]
            us = []
            for h in range(H):
                cc = 2 * h
                s0 = xs[cc] + xr_b[e, pl.ds(cc * 16, 16)]
                s1 = xs[cc + 1] + xr_b[e, pl.ds((cc + 1) * 16, 16)]
                t0 = jnp.maximum(s0, 0.2 * s0) * avs[cc]
                t1 = jnp.maximum(s1, 0.2 * s1) * avs[cc + 1]
                us.append(t0 + t1)
            als = [jnp.sum(u) for u in us]
            wvs = [jnp.exp(jnp.full((16,), a, jnp.float32)) for a in als]
            wrow = jnp.zeros((16,), jnp.float32)
            for h in range(H):
                wrow = jnp.where(iota16 == h, wvs[h], wrow)
            for cc in range(8):
                msg_b[e, pl.ds(cc * 16, 16)] = xs[cc] * wvs[cc // 2]
            msg_b[e, pl.ds(128, 16)] = wrow

    def process_pair(cp, c, i, wrap):
        for b in range(2):
            row = 2 * i + b
            pltpu.make_async_copy(xl_hbm.at[schunk[cp].at[row]], xl_v[b],
                                  gsem_l[b]).wait()
            pltpu.make_async_copy(xr_hbm.at[dchunk[cp].at[row]], xr_v[b],
                                  gsem_r[b]).wait()
            pltpu.make_async_copy(msg_v[b], acc_sh.at[dchunk[cp].at[row]],
                                  ssem[b]).wait()
            compute_block(xl_v[b], xr_v[b], msg_v[b])
            pltpu.async_copy(msg_v[b], acc_sh.at[dchunk[cp].at[row]],
                             ssem[b], add=True)
            if not wrap:
                nrow = row + 2
                pltpu.async_copy(xl_hbm.at[schunk[cp].at[nrow]], xl_v[b],
                                 gsem_l[b])
                pltpu.async_copy(xr_hbm.at[dchunk[cp].at[nrow]], xr_v[b],
                                 gsem_r[b])
            else:
                pltpu.async_copy(xl_hbm.at[schunk[1 - cp].at[b]], xl_v[b],
                                 gsem_l[b])
                pltpu.async_copy(xr_hbm.at[dchunk[1 - cp].at[b]], xr_v[b],
                                 gsem_r[b])

    pltpu.sync_copy(src_hbm.at[tid, 0], schunk[0])
    pltpu.sync_copy(dst_hbm.at[tid, 0], dchunk[0])
    for b in range(2):
        pltpu.async_copy(xl_hbm.at[schunk[0].at[b]], xl_v[b], gsem_l[b])
        pltpu.async_copy(xr_hbm.at[dchunk[0].at[b]], xr_v[b], gsem_r[b])
        pltpu.async_copy(msg_v[b], acc_sh.at[dchunk[0].at[b]], ssem[b],
                         add=True)

    def chunk_pair(q, carry):
        for cp in range(2):
            c = 2 * q + cp
            process_pair(cp, c, 0, False)
            pltpu.sync_copy(src_hbm.at[tid, c + 1], schunk[1 - cp])
            pltpu.sync_copy(dst_hbm.at[tid, c + 1], dchunk[1 - cp])

            def inner(i, carry2):
                process_pair(cp, c, i, False)
                return carry2

            lax.fori_loop(1, CHUNK // 2 - 1, inner, 0)
            process_pair(cp, c, CHUNK // 2 - 1, True)
        return carry

    lax.fori_loop(0, NCHUNK // 2, chunk_pair, 0)

    for b in range(2):
        pltpu.make_async_copy(xl_hbm.at[schunk[0].at[b]], xl_v[b],
                              gsem_l[b]).wait()
        pltpu.make_async_copy(xr_hbm.at[dchunk[0].at[b]], xr_v[b],
                              gsem_r[b]).wait()
        pltpu.make_async_copy(msg_v[b], acc_sh.at[dchunk[0].at[b]],
                              ssem[b]).wait()
    plsc.subcore_barrier()

    pltpu.sync_copy(acc_sh.at[pl.ds(base, ROWS_PER_TILE)],
                    out_hbm.at[cid, pl.ds(base, ROWS_PER_TILE)])


def _post_body(acc_ref, res_ref, msel_ref, bias_ref, gam_ref, beta_ref, out_ref):
    a = acc_ref[0] + acc_ref[1]
    msg = a[:, :OUT]
    scale = jnp.dot(a, msel_ref[...], preferred_element_type=jnp.float32) + 1e-16
    out = msg / scale + res_ref[...] + bias_ref[...]
    mu = jnp.mean(out, axis=1, keepdims=True)
    var = jnp.mean((out - mu) * (out - mu), axis=1, keepdims=True)
    out = (out - mu) * lax.rsqrt(var + 1e-5) * gam_ref[...] + beta_ref[...]
    out_ref[...] = 0.5 * out * (1.0 + jnp.tanh(
        np.sqrt(2.0 / np.pi) * (out + 0.044715 * out * out * out)))


def kernel(x_in, edge_index, W_l, W_r, att, bias, W_res, ln_gamma, ln_beta):
    x_pad = jnp.concatenate(
        [x_in, jnp.zeros((NPAD - N, IN), jnp.float32)], axis=0)

    mm_rows = 2504
    xl_tab, xr_tab, res = pl.pallas_call(
        _mm_body,
        grid=(NPAD // mm_rows,),
        in_specs=[
            pl.BlockSpec((mm_rows, IN), lambda i: (i, 0)),
            pl.BlockSpec((IN, OUT), lambda i: (0, 0)),
            pl.BlockSpec((IN, OUT), lambda i: (0, 0)),
            pl.BlockSpec((IN, OUT), lambda i: (0, 0)),
        ],
        out_specs=[
            pl.BlockSpec((mm_rows, OUT), lambda i: (i, 0)),
            pl.BlockSpec((mm_rows, OUT), lambda i: (i, 0)),
            pl.BlockSpec((mm_rows, OUT), lambda i: (i, 0)),
        ],
        out_shape=[
            jax.ShapeDtypeStruct((NPAD, OUT), jnp.float32),
            jax.ShapeDtypeStruct((NPAD, OUT), jnp.float32),
            jax.ShapeDtypeStruct((NPAD, OUT), jnp.float32),
        ],
    )(x_pad, W_l, W_r, W_res)

    ei = edge_index.astype(jnp.int32)
    sl = jnp.arange(N, dtype=jnp.int32)
    npad_edges = EPAD - ei.shape[1] - N
    padv = jnp.full((npad_edges,), N, jnp.int32)
    dummy_chunk = jnp.full((NW, 1, CHUNK, EB), N, jnp.int32)
    src = jnp.concatenate([ei[0], sl, padv]).reshape(NW, NCHUNK, CHUNK, EB)
    dst = jnp.concatenate([ei[1], sl, padv]).reshape(NW, NCHUNK, CHUNK, EB)
    src = jnp.concatenate([src, dummy_chunk], axis=1)
    dst = jnp.concatenate([dst, dummy_chunk], axis=1)
    att_flat = att.reshape(H * C)

    sc_edge = functools.partial(
        pl.kernel,
        mesh=plsc.VectorSubcoreMesh(core_axis_name="c", subcore_axis_name="s"),
        out_type=jax.ShapeDtypeStruct((NC, NPAD, ROWW), jnp.float32),
        scratch_types=[
            [pltpu.VMEM((CHUNK, EB), jnp.int32)] * 2,
            [pltpu.VMEM((CHUNK, EB), jnp.int32)] * 2,
            [pltpu.VMEM((EB, IN), jnp.float32)] * 2,
            [pltpu.VMEM((EB, IN), jnp.float32)] * 2,
            [pltpu.VMEM((EB, ROWW), jnp.float32)] * 2,
            pltpu.VMEM((IN,), jnp.float32),
            pltpu.VMEM_SHARED((NPAD, ROWW), jnp.float32),
            [pltpu.SemaphoreType.DMA] * 2,
            [pltpu.SemaphoreType.DMA] * 2,
            [pltpu.SemaphoreType.DMA] * 2,
        ],
        compiler_params=pltpu.CompilerParams(
            use_tc_tiling_on_sc=False, needs_layout_passes=False),
    )(_sc_edge_body)
    acc = sc_edge(xl_tab, xr_tab, src, dst, att_flat)

    msel = np.zeros((ROWW, OUT), np.float32)
    for h in range(H):
        msel[OUT + h, h * C:(h + 1) * C] = 1.0
    msel = jnp.asarray(msel)
    post_rows = 2000
    out = pl.pallas_call(
        _post_body,
        grid=(N // post_rows,),
        in_specs=[
            pl.BlockSpec((NC, post_rows, ROWW), lambda i: (0, i, 0)),
            pl.BlockSpec((post_rows, OUT), lambda i: (i, 0)),
            pl.BlockSpec((ROWW, OUT), lambda i: (0, 0)),
            pl.BlockSpec((1, OUT), lambda i: (0, 0)),
            pl.BlockSpec((1, OUT), lambda i: (0, 0)),
            pl.BlockSpec((1, OUT), lambda i: (0, 0)),
        ],
        out_specs=pl.BlockSpec((post_rows, OUT), lambda i: (i, 0)),
        out_shape=jax.ShapeDtypeStruct((N, OUT), jnp.float32),
    )(acc, res[:N], msel, bias.reshape(1, OUT), ln_gamma.reshape(1, OUT),
      ln_beta.reshape(1, OUT))
    return out
